# baseline (speedup 1.0000x reference)
import jax
import jax.numpy as jnp
from jax import lax
from jax.experimental import pallas as pl
from jax.experimental.pallas import tpu as pltpu

N = 50000
D = 256
S = 512
R = 5000
NB = N // R
WIN = 128


def _body(x_ref, bseg_ref, bsm_ref, wt_ref, bias_ref, out_ref, m_ref, z_ref):
    i = pl.program_id(0)

    @pl.when(i == 0)
    def _init():
        m_ref[0] = -jnp.inf
        z_ref[0] = 0.0
        out_ref[...] = jnp.zeros_like(out_ref)

    xb = x_ref[...]
    srow = lax.dot_general(wt_ref[...], xb, (((1,), (1,)), ((), ())),
                           preferred_element_type=jnp.float32)
    srow = srow + bias_ref[0, 0]

    m_old = m_ref[0]
    m_new = jnp.maximum(m_old, jnp.max(srow))
    p = jnp.exp(srow - m_new)
    z_ref[0] = z_ref[0] * jnp.exp(m_old - m_new) + jnp.sum(p)
    m_ref[0] = m_new

    @pl.when(jnp.logical_and(i > 0, m_new > m_old))
    def _rescale():
        out_ref[...] = out_ref[...] * jnp.exp(m_old - m_new)

    seg = bseg_ref[0, 0, :]
    base8 = jnp.minimum((bsm_ref[0, 0, 0] // 8) * 8, S - WIN)
    hi = bsm_ref[0, 0, R - 1]
    in_window = hi - base8 < WIN

    @pl.when(in_window)
    def _fast():
        offs = seg - base8
        wmat = jnp.where(
            lax.broadcasted_iota(jnp.int32, (WIN, R), 0) == offs[None, :],
            p, 0.0)
        contrib = jnp.dot(wmat, xb, preferred_element_type=jnp.float32)
        out_ref[pl.ds(base8, WIN), :] = out_ref[pl.ds(base8, WIN), :] + contrib

    @pl.when(jnp.logical_not(in_window))
    def _slow():
        wmat = jnp.where(
            lax.broadcasted_iota(jnp.int32, (S, R), 0) == seg[None, :],
            p, 0.0)
        contrib = jnp.dot(wmat, xb, preferred_element_type=jnp.float32)
        out_ref[...] = out_ref[...] + contrib

    @pl.when(i == NB - 1)
    def _fin():
        out_ref[...] = out_ref[...] * (1.0 / z_ref[0])


def kernel(x, batch, W, b):
    b3 = batch.astype(jnp.int32).reshape(NB, 1, R)
    return pl.pallas_call(
        _body,
        grid=(NB,),
        in_specs=[
            pl.BlockSpec((R, D), lambda i: (i, 0)),
            pl.BlockSpec((1, 1, R), lambda i: (i, 0, 0)),
            pl.BlockSpec((1, 1, R), lambda i: (i, 0, 0),
                         memory_space=pltpu.SMEM),
            pl.BlockSpec((1, D), lambda i: (0, 0)),
            pl.BlockSpec((1, 1), lambda i: (0, 0)),
        ],
        out_specs=pl.BlockSpec((S, D), lambda i: (0, 0)),
        out_shape=jax.ShapeDtypeStruct((S, D), jnp.float32),
        scratch_shapes=[pltpu.SMEM((1,), jnp.float32),
                        pltpu.SMEM((1,), jnp.float32)],
    )(x, b3, b3, W.reshape(1, D), b.reshape(1, 1))
